# baseline (device time: 28477 ns/iter reference)
import jax
import jax.numpy as jnp
from jax import lax
from jax.experimental import pallas as pl
from jax.experimental.pallas import tpu as pltpu

NC = 4


def kernel(x, router, W1, W2):
    T_loc, D = x.shape
    E_loc, _, F = W1.shape
    FC = F // NC

    def body(x_ref, r_ref, w1_ref, w2_ref, out_ref,
             xsend, xin, rpeer, wsend, win, psend, pin,
             wb1, wb2, st1, st2, sems, csem1, csem2):
        my_x = lax.axis_index("x")
        my_y = lax.axis_index("y")
        my_z = lax.axis_index("z")
        peer = (1 - my_x, my_y, my_z)

        barrier = pltpu.get_barrier_semaphore()
        pl.semaphore_signal(barrier, inc=1, device_id=peer,
                            device_id_type=pl.DeviceIdType.MESH)
        pl.semaphore_wait(barrier, 1)

        xsend[:, :] = x_ref[:, :].astype(jnp.bfloat16)
        rdma_x = pltpu.make_async_remote_copy(
            src_ref=xsend, dst_ref=xin,
            send_sem=sems.at[0], recv_sem=sems.at[1],
            device_id=peer, device_id_type=pl.DeviceIdType.MESH,
        )
        rdma_x.start()
        rdma_r = pltpu.make_async_remote_copy(
            src_ref=r_ref, dst_ref=rpeer,
            send_sem=sems.at[2], recv_sem=sems.at[3],
            device_id=peer, device_id_type=pl.DeviceIdType.MESH,
        )
        rdma_r.start()

        cp1, cp2 = [], []
        for k in range(NC):
            c1 = pltpu.make_async_copy(
                w1_ref.at[:, :, pl.ds(k * FC, FC)], st1.at[k], csem1.at[k])
            c2 = pltpu.make_async_copy(
                w2_ref.at[:, pl.ds(k * FC, FC), :], st2.at[k], csem2.at[k])
            c1.start()
            c2.start()
            cp1.append(c1)
            cp2.append(c2)

        rdma_r.wait()
        xl = x_ref[:, :]
        gl = jnp.dot(xl, r_ref[:, :], preferred_element_type=jnp.float32)
        gr = jnp.dot(xl, rpeer[:, :], preferred_element_type=jnp.float32)

        g0, g1 = gl[:, 0:1], gl[:, 1:2]
        g2, g3 = gr[:, 0:1], gr[:, 1:2]
        a = jnp.maximum(g0, g1)
        b = jnp.minimum(g0, g1)
        c = jnp.maximum(g2, g3)
        d = jnp.minimum(g2, g3)
        m1 = jnp.maximum(a, c)
        m2 = jnp.where(a >= c, jnp.maximum(b, c), jnp.maximum(d, a))

        def wgt(g):
            return jnp.where(g >= m2, jnp.exp(g - m1), 0.0)

        t0, t1, t2, t3 = wgt(g0), wgt(g1), wgt(g2), wgt(g3)
        z = t0 + t1 + t2 + t3
        w_mine = [t0 / z, t1 / z]

        wsend[:, :] = jnp.concatenate([t2 / z, t3 / z], axis=1)
        rdma_w = pltpu.make_async_remote_copy(
            src_ref=wsend, dst_ref=win,
            send_sem=sems.at[4], recv_sem=sems.at[5],
            device_id=peer, device_id_type=pl.DeviceIdType.MESH,
        )
        rdma_w.start()

        xp = None
        accs = [jnp.zeros((T_loc, D), jnp.float32) for _ in range(E_loc)]
        for k in range(NC):
            cp1[k].wait()
            cp2[k].wait()
            w1c, w2c = [], []
            for e in range(E_loc):
                w1ce = st1[k, e, :, :].astype(jnp.bfloat16)
                w2ce = st2[k, e, :, :].astype(jnp.bfloat16)
                wb1[e, :, pl.ds(k * FC, FC)] = w1ce
                wb2[e, pl.ds(k * FC, FC), :] = w2ce
                w1c.append(w1ce)
                w2c.append(w2ce)
            if k == 0:
                rdma_x.wait()
                rdma_w.wait()
                xp = xin[:, :]
            for e in range(E_loc):
                h = jnp.dot(xp, w1c[e], preferred_element_type=jnp.float32)
                h = jnp.maximum(h, 0.0).astype(jnp.bfloat16)
                accs[e] = accs[e] + jnp.dot(
                    h, w2c[e], preferred_element_type=jnp.float32)
        acc_peer = accs[0] * win[:, 0:1] + accs[1] * win[:, 1:2]

        psend[:, :] = acc_peer.astype(jnp.bfloat16)
        rdma_p = pltpu.make_async_remote_copy(
            src_ref=psend, dst_ref=pin,
            send_sem=sems.at[6], recv_sem=sems.at[7],
            device_id=peer, device_id_type=pl.DeviceIdType.MESH,
        )
        rdma_p.start()

        xbl = xl.astype(jnp.bfloat16)
        acc_my = jnp.zeros((T_loc, D), jnp.float32)
        for e in range(E_loc):
            h = jnp.dot(xbl, wb1[e, :, :], preferred_element_type=jnp.float32)
            h = jnp.maximum(h, 0.0).astype(jnp.bfloat16)
            o = jnp.dot(h, wb2[e, :, :], preferred_element_type=jnp.float32)
            acc_my = acc_my + o * w_mine[e]

        rdma_p.wait()
        out_ref[:, :] = acc_my + pin[:, :].astype(jnp.float32)

    return pl.pallas_call(
        body,
        out_shape=jax.ShapeDtypeStruct((T_loc, D), jnp.float32),
        in_specs=[
            pl.BlockSpec(memory_space=pltpu.VMEM),
            pl.BlockSpec(memory_space=pltpu.VMEM),
            pl.BlockSpec(memory_space=pltpu.MemorySpace.HBM),
            pl.BlockSpec(memory_space=pltpu.MemorySpace.HBM),
        ],
        out_specs=pl.BlockSpec(memory_space=pltpu.VMEM),
        scratch_shapes=[
            pltpu.VMEM((T_loc, D), jnp.bfloat16),
            pltpu.VMEM((T_loc, D), jnp.bfloat16),
            pltpu.VMEM((D, E_loc), jnp.float32),
            pltpu.VMEM((T_loc, E_loc), jnp.float32),
            pltpu.VMEM((T_loc, E_loc), jnp.float32),
            pltpu.VMEM((T_loc, D), jnp.bfloat16),
            pltpu.VMEM((T_loc, D), jnp.bfloat16),
            pltpu.VMEM((E_loc, D, F), jnp.bfloat16),
            pltpu.VMEM((E_loc, F, D), jnp.bfloat16),
            pltpu.VMEM((NC, E_loc, D, FC), jnp.float32),
            pltpu.VMEM((NC, E_loc, FC, D), jnp.float32),
            pltpu.SemaphoreType.DMA((8,)),
            pltpu.SemaphoreType.DMA((NC,)),
            pltpu.SemaphoreType.DMA((NC,)),
        ],
        compiler_params=pltpu.CompilerParams(collective_id=0),
    )(x, router, W1, W2)


# device time: 28081 ns/iter; 1.0141x vs baseline; 1.0141x over previous
import jax
import jax.numpy as jnp
from jax import lax
from jax.experimental import pallas as pl
from jax.experimental.pallas import tpu as pltpu


def kernel(x, router, W1, W2):
    T_loc, D = x.shape
    E_loc, _, F = W1.shape

    def body(x_ref, r_ref, w1_ref, w2_ref, out_ref,
             xsend, xin, rpeer, wsend, win, psend, pin,
             w1v, w2v, sems, csem):
        my_x = lax.axis_index("x")
        my_y = lax.axis_index("y")
        my_z = lax.axis_index("z")
        peer = (1 - my_x, my_y, my_z)

        cp1 = pltpu.make_async_copy(w1_ref, w1v, csem.at[0])
        cp2 = pltpu.make_async_copy(w2_ref, w2v, csem.at[1])
        cp1.start()
        cp2.start()

        barrier = pltpu.get_barrier_semaphore()
        pl.semaphore_signal(barrier, inc=1, device_id=peer,
                            device_id_type=pl.DeviceIdType.MESH)
        pl.semaphore_wait(barrier, 1)

        xsend[:, :] = x_ref[:, :].astype(jnp.bfloat16)
        rdma_x = pltpu.make_async_remote_copy(
            src_ref=xsend, dst_ref=xin,
            send_sem=sems.at[0], recv_sem=sems.at[1],
            device_id=peer, device_id_type=pl.DeviceIdType.MESH,
        )
        rdma_x.start()
        rdma_r = pltpu.make_async_remote_copy(
            src_ref=r_ref, dst_ref=rpeer,
            send_sem=sems.at[2], recv_sem=sems.at[3],
            device_id=peer, device_id_type=pl.DeviceIdType.MESH,
        )
        rdma_r.start()
        rdma_r.wait()

        xl = x_ref[:, :]
        gl = jnp.dot(xl, r_ref[:, :], preferred_element_type=jnp.float32)
        gr = jnp.dot(xl, rpeer[:, :], preferred_element_type=jnp.float32)

        g0, g1 = gl[:, 0:1], gl[:, 1:2]
        g2, g3 = gr[:, 0:1], gr[:, 1:2]
        a = jnp.maximum(g0, g1)
        b = jnp.minimum(g0, g1)
        c = jnp.maximum(g2, g3)
        d = jnp.minimum(g2, g3)
        m1 = jnp.maximum(a, c)
        m2 = jnp.where(a >= c, jnp.maximum(b, c), jnp.maximum(d, a))

        def wgt(g):
            return jnp.where(g >= m2, jnp.exp(g - m1), 0.0)

        t0, t1, t2, t3 = wgt(g0), wgt(g1), wgt(g2), wgt(g3)
        z = t0 + t1 + t2 + t3
        w_mine = [t0 / z, t1 / z]

        wsend[:, :] = jnp.concatenate([t2 / z, t3 / z], axis=1)
        rdma_w = pltpu.make_async_remote_copy(
            src_ref=wsend, dst_ref=win,
            send_sem=sems.at[4], recv_sem=sems.at[5],
            device_id=peer, device_id_type=pl.DeviceIdType.MESH,
        )
        rdma_w.start()

        cp1.wait()
        cp2.wait()
        w1b = [w1v[e, :, :].astype(jnp.bfloat16) for e in range(E_loc)]
        w2b = [w2v[e, :, :].astype(jnp.bfloat16) for e in range(E_loc)]

        rdma_x.wait()
        rdma_w.wait()
        xp = xin[:, :]
        acc_peer = jnp.zeros((T_loc, D), jnp.float32)
        for e in range(E_loc):
            h = jnp.dot(xp, w1b[e], preferred_element_type=jnp.float32)
            h = jnp.maximum(h, 0.0).astype(jnp.bfloat16)
            o = jnp.dot(h, w2b[e], preferred_element_type=jnp.float32)
            acc_peer = acc_peer + o * win[:, e:e + 1]

        psend[:, :] = acc_peer.astype(jnp.bfloat16)
        rdma_p = pltpu.make_async_remote_copy(
            src_ref=psend, dst_ref=pin,
            send_sem=sems.at[6], recv_sem=sems.at[7],
            device_id=peer, device_id_type=pl.DeviceIdType.MESH,
        )
        rdma_p.start()

        xbl = xl.astype(jnp.bfloat16)
        acc_my = jnp.zeros((T_loc, D), jnp.float32)
        for e in range(E_loc):
            h = jnp.dot(xbl, w1b[e], preferred_element_type=jnp.float32)
            h = jnp.maximum(h, 0.0).astype(jnp.bfloat16)
            o = jnp.dot(h, w2b[e], preferred_element_type=jnp.float32)
            acc_my = acc_my + o * w_mine[e]

        rdma_p.wait()
        out_ref[:, :] = acc_my + pin[:, :].astype(jnp.float32)

    return pl.pallas_call(
        body,
        out_shape=jax.ShapeDtypeStruct((T_loc, D), jnp.float32),
        in_specs=[
            pl.BlockSpec(memory_space=pltpu.VMEM),
            pl.BlockSpec(memory_space=pltpu.VMEM),
            pl.BlockSpec(memory_space=pltpu.MemorySpace.HBM),
            pl.BlockSpec(memory_space=pltpu.MemorySpace.HBM),
        ],
        out_specs=pl.BlockSpec(memory_space=pltpu.VMEM),
        scratch_shapes=[
            pltpu.VMEM((T_loc, D), jnp.bfloat16),
            pltpu.VMEM((T_loc, D), jnp.bfloat16),
            pltpu.VMEM((D, E_loc), jnp.float32),
            pltpu.VMEM((T_loc, E_loc), jnp.float32),
            pltpu.VMEM((T_loc, E_loc), jnp.float32),
            pltpu.VMEM((T_loc, D), jnp.bfloat16),
            pltpu.VMEM((T_loc, D), jnp.bfloat16),
            pltpu.VMEM((E_loc, D, F), jnp.float32),
            pltpu.VMEM((E_loc, F, D), jnp.float32),
            pltpu.SemaphoreType.DMA((8,)),
            pltpu.SemaphoreType.DMA((2,)),
        ],
        compiler_params=pltpu.CompilerParams(collective_id=0),
    )(x, router, W1, W2)
